# Initial kernel scaffold; baseline (speedup 1.0000x reference)
import jax
import jax.numpy as jnp
from jax.experimental import pallas as pl


def kernel(x, W_in, b_in, W1, g1, be1, W2, g2, be2, W3, g3, be3, W4, g4, be4):
    raise NotImplementedError("write your pallas kernel here")


# speedup vs baseline: 4.4333x; 4.4333x over previous
import jax
import jax.numpy as jnp
from jax.experimental import pallas as pl
from jax.experimental.pallas import tpu as pltpu

_K = 16
_QB = 128
_BIG = 1e30


def _graph_stage(coords, coords_t, fk, qidx, W, gam, bet, *,
                 Nk, Nq, Ck, Cout, Cg, stage1_Win=None, stage1_bin=None):
    B = coords.shape[0]
    QBN = Nq // _QB
    stage1 = fk is None
    gather_q = qidx is not None

    def body(*args):
        it = iter(args)
        c_ref = next(it)
        ct_ref = next(it)
        fk_ref = None if stage1 else next(it)
        qidx_ref = next(it) if gather_q else None
        if stage1:
            Win_ref = next(it)
            bin_ref = next(it)
        W_ref = next(it)
        gam_ref = next(it)
        bet_ref = next(it)
        fout_ref = next(it)
        cq_ref = next(it) if gather_q else None
        d_ref = next(it)
        ksq_ref = next(it)
        f1_ref = next(it) if stage1 else None
        maxb_ref = next(it)
        minb_ref = next(it)
        maxq_ref = next(it)
        minq_ref = next(it)
        sum_ref = next(it)
        sq_ref = next(it)

        q = pl.program_id(1)
        coords_v = c_ref[0]

        @pl.when(q == 0)
        def _init():
            ct = ct_ref[0]
            ksq_ref[...] = jnp.sum(ct * ct, axis=1, keepdims=True)
            sum_ref[...] = jnp.zeros_like(sum_ref)
            sq_ref[...] = jnp.zeros_like(sq_ref)
            if stage1:
                f1_ref[...] = jnp.dot(
                    Win_ref[...], coords_v,
                    preferred_element_type=jnp.float32,
                    precision=jax.lax.Precision.DEFAULT) + bin_ref[...]

        fk_v = f1_ref[...] if stage1 else fk_ref[0]
        qsl = pl.ds(q * _QB, _QB)
        iota0 = jax.lax.broadcasted_iota(jnp.int32, (Nk, _QB), 0)

        if gather_q:
            qi = qidx_ref[0, :, qsl]
            ohq = (iota0 == qi).astype(jnp.float32)
            xq = jnp.dot(coords_v, ohq, preferred_element_type=jnp.float32, precision=jax.lax.Precision.HIGHEST)
            fq = jnp.dot(fk_v, ohq, preferred_element_type=jnp.float32, precision=jax.lax.Precision.HIGHEST)
            cq_ref[0] = xq
        else:
            xq = c_ref[0, :, qsl]
            fq = f1_ref[:, qsl] if stage1 else fk_ref[0, :, qsl]

        qsq = jnp.sum(xq * xq, axis=0, keepdims=True)
        dotp = jax.lax.dot_general(coords_v, xq, (((0,), (0,)), ((), ())),
                                   preferred_element_type=jnp.float32,
                                   precision=jax.lax.Precision.DEFAULT)
        d_ref[...] = jnp.maximum((ksq_ref[...] + qsq) - 2.0 * dotp, 0.0)

        maxq_ref[...] = jnp.full_like(maxq_ref, -_BIG)
        minq_ref[...] = jnp.full_like(minq_ref, _BIG)

        def ext(t, carry):
            d = d_ref[...]
            m = jnp.min(d, axis=0, keepdims=True)
            am = jnp.min(jnp.where(d == m, iota0, Nk), axis=0, keepdims=True)
            eq = iota0 == am
            d_ref[...] = jnp.where(eq, _BIG, d)
            g = jnp.dot(fk_v, eq.astype(jnp.float32),
                        preferred_element_type=jnp.float32,
                        precision=jax.lax.Precision.HIGHEST)
            cat = jnp.concatenate([g - fq, fq], axis=0)
            out = jnp.dot(W_ref[...], cat,
                          preferred_element_type=jnp.float32,
                          precision=jax.lax.Precision.DEFAULT)
            maxq_ref[...] = jnp.maximum(maxq_ref[...], out)
            minq_ref[...] = jnp.minimum(minq_ref[...], out)
            sum_ref[...] += out
            sq_ref[...] += out * out
            return carry

        jax.lax.fori_loop(0, _K, ext, 0)
        maxb_ref[:, qsl] = maxq_ref[...]
        minb_ref[:, qsl] = minq_ref[...]

        @pl.when(q == QBN - 1)
        def _fin():
            s = jnp.sum(sum_ref[...], axis=1, keepdims=True)
            s2 = jnp.sum(sq_ref[...], axis=1, keepdims=True)
            r = jax.lax.broadcasted_iota(jnp.int32, (Cout, Cout), 0) // Cg
            c = jax.lax.broadcasted_iota(jnp.int32, (Cout, Cout), 1) // Cg
            M = (r == c).astype(jnp.float32)
            gs = jnp.dot(M, s, preferred_element_type=jnp.float32, precision=jax.lax.Precision.HIGHEST)
            gs2 = jnp.dot(M, s2, preferred_element_type=jnp.float32, precision=jax.lax.Precision.HIGHEST)
            n = jnp.float32(Cg * Nq * _K)
            mean = gs / n
            var = gs2 / n - mean * mean
            scale = gam_ref[...] / jnp.sqrt(var + 1e-5)
            shift = bet_ref[...] - mean * scale
            pre = jnp.where(scale >= 0.0, maxb_ref[...], minb_ref[...])
            pre = pre * scale + shift
            fout_ref[0] = jnp.where(pre >= 0.0, pre, 0.2 * pre)

    full = lambda b, q: (b, 0, 0)
    wspec = lambda a: pl.BlockSpec(a.shape, lambda b, q: (0,) * a.ndim)
    in_specs = [pl.BlockSpec((1, 3, Nk), full),
                pl.BlockSpec((1, Nk, 3), full)]
    inputs = [coords, coords_t]
    if not stage1:
        in_specs.append(pl.BlockSpec((1, Ck, Nk), full))
        inputs.append(fk)
    if gather_q:
        in_specs.append(pl.BlockSpec((1, 1, Nq), full))
        inputs.append(qidx)
    if stage1:
        in_specs += [wspec(stage1_Win), wspec(stage1_bin)]
        inputs += [stage1_Win, stage1_bin]
    in_specs += [wspec(W), wspec(gam), wspec(bet)]
    inputs += [W, gam, bet]

    out_shape = [jax.ShapeDtypeStruct((B, Cout, Nq), jnp.float32)]
    out_specs = [pl.BlockSpec((1, Cout, Nq), full)]
    if gather_q:
        out_shape.append(jax.ShapeDtypeStruct((B, 3, Nq), jnp.float32))
        out_specs.append(pl.BlockSpec((1, 3, _QB), lambda b, q: (b, 0, q)))

    scratch = [pltpu.VMEM((Nk, _QB), jnp.float32),
               pltpu.VMEM((Nk, 1), jnp.float32)]
    if stage1:
        scratch.append(pltpu.VMEM((Ck, Nk), jnp.float32))
    scratch += [pltpu.VMEM((Cout, Nq), jnp.float32),
                pltpu.VMEM((Cout, Nq), jnp.float32),
                pltpu.VMEM((Cout, _QB), jnp.float32),
                pltpu.VMEM((Cout, _QB), jnp.float32),
                pltpu.VMEM((Cout, _QB), jnp.float32),
                pltpu.VMEM((Cout, _QB), jnp.float32)]

    res = pl.pallas_call(
        body,
        grid=(B, QBN),
        in_specs=in_specs,
        out_specs=out_specs,
        out_shape=out_shape,
        scratch_shapes=scratch,
        compiler_params=pltpu.CompilerParams(
            dimension_semantics=("arbitrary", "arbitrary")),
    )(*inputs)
    return res if gather_q else res[0]


def _fps(coords_t, S):
    _, N, B = coords_t.shape

    def body(c_ref, idx_ref, dists_ref, far_ref):
        X, Y, Z = c_ref[0], c_ref[1], c_ref[2]
        dists_ref[...] = jnp.full((N, B), 1e10, jnp.float32)
        far_ref[...] = jnp.zeros((1, B), jnp.int32)
        iota0 = jax.lax.broadcasted_iota(jnp.int32, (N, B), 0)

        def step(i, carry):
            far = far_ref[...]
            idx_ref[pl.ds(i, 1), :] = far
            oh = (iota0 == far).astype(jnp.float32)
            cx = jnp.sum(X * oh, axis=0, keepdims=True)
            cy = jnp.sum(Y * oh, axis=0, keepdims=True)
            cz = jnp.sum(Z * oh, axis=0, keepdims=True)
            dx, dy, dz = X - cx, Y - cy, Z - cz
            dnew = (dx * dx + dy * dy) + dz * dz
            dists = jnp.minimum(dists_ref[...], dnew)
            dists_ref[...] = dists
            m = jnp.max(dists, axis=0, keepdims=True)
            far_ref[...] = jnp.min(jnp.where(dists == m, iota0, N),
                                   axis=0, keepdims=True)
            return carry

        jax.lax.fori_loop(0, S, step, 0)

    return pl.pallas_call(
        body,
        out_shape=jax.ShapeDtypeStruct((S, B), jnp.int32),
        scratch_shapes=[pltpu.VMEM((N, B), jnp.float32),
                        pltpu.VMEM((1, B), jnp.int32)],
    )(coords_t)


def kernel(x, W_in, b_in, W1, g1, be1, W2, g2, be2, W3, g3, be3, W4, g4, be4):
    B = x.shape[0]
    col = lambda v: v.reshape(-1, 1).astype(jnp.float32)

    xt = jnp.transpose(x, (0, 2, 1))
    f2 = _graph_stage(x, xt, None, None, W1, col(g1), col(be1),
                      Nk=2048, Nq=2048, Ck=8, Cout=32, Cg=8,
                      stage1_Win=W_in, stage1_bin=col(b_in))

    idx1 = _fps(jnp.transpose(x, (1, 2, 0)), 512)
    qidx1 = jnp.transpose(idx1).reshape(B, 1, 512)

    f3, coor_q = _graph_stage(x, xt, f2, qidx1, W2, col(g2), col(be2),
                              Nk=2048, Nq=512, Ck=32, Cout=64, Cg=16)

    cqt = jnp.transpose(coor_q, (0, 2, 1))
    f4 = _graph_stage(coor_q, cqt, f3, None, W3, col(g3), col(be3),
                      Nk=512, Nq=512, Ck=64, Cout=64, Cg=16)

    idx2 = _fps(jnp.transpose(coor_q, (1, 2, 0)), 128)
    qidx2 = jnp.transpose(idx2).reshape(B, 1, 128)

    f5, coor_out = _graph_stage(coor_q, cqt, f4, qidx2, W4,
                                col(g4), col(be4),
                                Nk=512, Nq=128, Ck=64, Cout=128, Cg=32)
    return coor_out, f5
